# baseline (speedup 1.0000x reference)
import functools

import jax
import jax.numpy as jnp
from jax import lax
from jax.experimental import pallas as pl
from jax.experimental.pallas import tpu as pltpu
from jax.experimental.pallas import tpu_sc as plsc

_VOCAB = 100000
_EMBED = 32
_BATCH = 1024
_CTX = 20

_NC = 2
_NS = 16
_NW = _NC * _NS
_B_PER_W = _BATCH // _NW
_IDX_PER_W = _B_PER_W * _CTX
_IDX_CHUNK = 128
_N_CHUNKS = _IDX_PER_W // _IDX_CHUNK
_LANES = 16


def _sc_gather_mean(idx_grouped, emb_table):
    mesh = plsc.VectorSubcoreMesh(core_axis_name="c", subcore_axis_name="s")

    @functools.partial(
        pl.kernel,
        mesh=mesh,
        out_type=jax.ShapeDtypeStruct((_BATCH, _EMBED), jnp.float32),
        compiler_params=pltpu.CompilerParams(use_tc_tiling_on_sc=False),
        scratch_types=[
            pltpu.VMEM((_N_CHUNKS, _IDX_CHUNK), jnp.int32),
            pltpu.VMEM((_IDX_PER_W, _EMBED), jnp.float32),
            pltpu.VMEM((_B_PER_W, _EMBED), jnp.float32),
            pltpu.SemaphoreType.DMA,
        ],
    )
    def body(idx_hbm, table_hbm, x_hbm, idx_v, rows_v, out_v, sem):
        wid = lax.axis_index("s") * _NC + lax.axis_index("c")
        pltpu.sync_copy(idx_hbm.at[wid], idx_v)
        copies = [
            pltpu.async_copy(
                table_hbm.at[idx_v.at[j]],
                rows_v.at[pl.ds(j * _IDX_CHUNK, _IDX_CHUNK)],
                sem,
            )
            for j in range(_N_CHUNKS)
        ]
        for c in copies:
            c.wait()

        scale = jnp.float32(1.0 / _CTX)

        def accum(i, carry):
            base = i * _CTX
            acc0 = rows_v[base, pl.ds(0, _LANES)]
            acc1 = rows_v[base, pl.ds(_LANES, _LANES)]
            for j in range(1, _CTX):
                acc0 = acc0 + rows_v[base + j, pl.ds(0, _LANES)]
                acc1 = acc1 + rows_v[base + j, pl.ds(_LANES, _LANES)]
            out_v[i, pl.ds(0, _LANES)] = acc0 * scale
            out_v[i, pl.ds(_LANES, _LANES)] = acc1 * scale
            return carry

        lax.fori_loop(0, _B_PER_W, accum, 0)
        pltpu.sync_copy(out_v, x_hbm.at[pl.ds(wid * _B_PER_W, _B_PER_W)])

    return body(idx_grouped, emb_table)


_TILE_V = 2048


def _mm_body(x_ref, wt_ref, o_ref):
    o_ref[...] = lax.dot_general(
        wt_ref[...], x_ref[...],
        (((0,), (1,)), ((), ())),
        preferred_element_type=jnp.float32,
    )


def _tc_project(x1, Wb):
    k = x1.shape[1]
    out_t = pl.pallas_call(
        _mm_body,
        grid=(pl.cdiv(_VOCAB, _TILE_V),),
        in_specs=[
            pl.BlockSpec((_BATCH, k), lambda i: (0, 0)),
            pl.BlockSpec((k, _TILE_V), lambda i: (0, i)),
        ],
        out_specs=pl.BlockSpec((_TILE_V, _BATCH), lambda i: (i, 0)),
        out_shape=jax.ShapeDtypeStruct((_VOCAB, _BATCH), jnp.float32),
    )(x1, Wb)
    return out_t.T


def kernel(inputs, emb_table, W, b):
    idx_grouped = inputs.astype(jnp.int32).reshape(_NW, _N_CHUNKS, _IDX_CHUNK)
    Wb = jnp.concatenate([W.T, b[None, :]], axis=0)
    emb_lin = lax.optimization_barrier(emb_table.reshape(_VOCAB * _EMBED))
    x = _sc_gather_mean(idx_grouped, emb_lin.reshape(_VOCAB, _EMBED))
    x1 = jnp.concatenate([x, jnp.ones((_BATCH, 1), jnp.float32)], axis=1)
    return _tc_project(x1, Wb)


# speedup vs baseline: 1.0071x; 1.0071x over previous
import functools

import jax
import jax.numpy as jnp
from jax import lax
from jax.experimental import pallas as pl
from jax.experimental.pallas import tpu as pltpu
from jax.experimental.pallas import tpu_sc as plsc

_VOCAB = 100000
_EMBED = 32
_BATCH = 1024
_CTX = 20

_NC = 2
_NS = 16
_NW = _NC * _NS
_B_PER_W = _BATCH // _NW
_IDX_PER_W = _B_PER_W * _CTX
_IDX_CHUNK = 128
_N_CHUNKS = _IDX_PER_W // _IDX_CHUNK
_LANES = 16


_ROW_PITCH = 128
_T_TILE = 2048


def _pad_body(in_ref, out_ref):
    out_ref[:, pl.ds(0, _EMBED)] = lax.transpose(in_ref[...], (1, 0))


def _stage_table(emb_t):
    return pl.pallas_call(
        _pad_body,
        grid=(pl.cdiv(_VOCAB, _T_TILE),),
        in_specs=[pl.BlockSpec((_EMBED, _T_TILE), lambda i: (0, i))],
        out_specs=pl.BlockSpec((_T_TILE, _ROW_PITCH), lambda i: (i, 0)),
        out_shape=jax.ShapeDtypeStruct((_VOCAB, _ROW_PITCH), jnp.float32),
    )(emb_t)


def _sc_gather_mean(idx_grouped, table_padded):
    mesh = plsc.VectorSubcoreMesh(core_axis_name="c", subcore_axis_name="s")

    @functools.partial(
        pl.kernel,
        mesh=mesh,
        out_type=jax.ShapeDtypeStruct((_BATCH, _EMBED), jnp.float32),
        compiler_params=pltpu.CompilerParams(use_tc_tiling_on_sc=False),
        scratch_types=[
            pltpu.VMEM((_N_CHUNKS, _IDX_CHUNK), jnp.int32),
            pltpu.VMEM((_IDX_PER_W, _ROW_PITCH), jnp.float32),
            pltpu.VMEM((_B_PER_W, _EMBED), jnp.float32),
            pltpu.SemaphoreType.DMA,
        ],
    )
    def body(idx_hbm, table_hbm, x_hbm, idx_v, rows_v, out_v, sem):
        wid = lax.axis_index("s") * _NC + lax.axis_index("c")
        pltpu.sync_copy(idx_hbm.at[wid], idx_v)
        copies = [
            pltpu.async_copy(
                table_hbm.at[idx_v.at[j]],
                rows_v.at[pl.ds(j * _IDX_CHUNK, _IDX_CHUNK)],
                sem,
            )
            for j in range(_N_CHUNKS)
        ]
        for c in copies:
            c.wait()

        scale = jnp.float32(1.0 / _CTX)

        def accum(i, carry):
            base = i * _CTX
            acc0 = rows_v[base, pl.ds(0, _LANES)]
            acc1 = rows_v[base, pl.ds(_LANES, _LANES)]
            for j in range(1, _CTX):
                acc0 = acc0 + rows_v[base + j, pl.ds(0, _LANES)]
                acc1 = acc1 + rows_v[base + j, pl.ds(_LANES, _LANES)]
            out_v[i, pl.ds(0, _LANES)] = acc0 * scale
            out_v[i, pl.ds(_LANES, _LANES)] = acc1 * scale
            return carry

        lax.fori_loop(0, _B_PER_W, accum, 0)
        pltpu.sync_copy(out_v, x_hbm.at[pl.ds(wid * _B_PER_W, _B_PER_W)])

    return body(idx_grouped, table_padded)


_TILE_V = 2048


def _mm_body(x_ref, wt_ref, o_ref):
    o_ref[...] = lax.dot_general(
        wt_ref[...], x_ref[...],
        (((0,), (1,)), ((), ())),
        preferred_element_type=jnp.float32,
    )


def _tc_project(x1, Wb):
    k = x1.shape[1]
    out_t = pl.pallas_call(
        _mm_body,
        grid=(pl.cdiv(_VOCAB, _TILE_V),),
        in_specs=[
            pl.BlockSpec((_BATCH, k), lambda i: (0, 0)),
            pl.BlockSpec((k, _TILE_V), lambda i: (0, i)),
        ],
        out_specs=pl.BlockSpec((_TILE_V, _BATCH), lambda i: (i, 0)),
        out_shape=jax.ShapeDtypeStruct((_VOCAB, _BATCH), jnp.float32),
    )(x1, Wb)
    return out_t.T


def kernel(inputs, emb_table, W, b):
    idx_grouped = inputs.astype(jnp.int32).reshape(_NW, _N_CHUNKS, _IDX_CHUNK)
    Wb = jnp.concatenate([W.T, b[None, :]], axis=0)
    x = _sc_gather_mean(idx_grouped, _stage_table(emb_table.T))
    x1 = jnp.concatenate([x, jnp.ones((_BATCH, 1), jnp.float32)], axis=1)
    return _tc_project(x1, Wb)


# speedup vs baseline: 1.0330x; 1.0257x over previous
import functools

import jax
import jax.numpy as jnp
from jax import lax
from jax.experimental import pallas as pl
from jax.experimental.pallas import tpu as pltpu
from jax.experimental.pallas import tpu_sc as plsc

_VOCAB = 100000
_EMBED = 32
_BATCH = 1024
_CTX = 20

_NC = 2
_NS = 16
_NW = _NC * _NS
_B_PER_W = _BATCH // _NW
_IDX_PER_W = _B_PER_W * _CTX
_IDX_CHUNK = 128
_N_CHUNKS = _IDX_PER_W // _IDX_CHUNK
_LANES = 16


_ROW_PITCH = 128
_T_TILE = 2048


_T_STEPS = pl.cdiv(_VOCAB, _T_TILE)
_T_TAIL = _VOCAB - (_T_STEPS - 1) * _T_TILE
_T_SLOTS = 4


def _pad_body(in_ref, o_hbm, buf, sem):
    i = pl.program_id(0)
    slot = lax.rem(i, _T_SLOTS)

    def _cp(step, slot_, rows):
        return pltpu.make_async_copy(
            buf.at[slot_, pl.ds(0, rows)],
            o_hbm.at[pl.ds(step * _T_TILE, rows)],
            sem.at[slot_],
        )

    @pl.when(i >= _T_SLOTS)
    def _():
        _cp(i - _T_SLOTS, slot, _T_TILE).wait()

    buf[slot, :, pl.ds(0, _EMBED)] = lax.transpose(in_ref[...], (1, 0))

    @pl.when(i < _T_STEPS - 1)
    def _():
        _cp(i, slot, _T_TILE).start()

    @pl.when(i == _T_STEPS - 1)
    def _():
        _cp(i, slot, _T_TAIL).start()
        for back in range(_T_SLOTS - 1, 0, -1):
            _cp(i - back, lax.rem(i - back, _T_SLOTS), _T_TILE).wait()
        _cp(i, slot, _T_TAIL).wait()


def _stage_table(emb_t):
    return pl.pallas_call(
        _pad_body,
        grid=(_T_STEPS,),
        in_specs=[pl.BlockSpec((_EMBED, _T_TILE), lambda i: (0, i))],
        out_specs=pl.BlockSpec(memory_space=pltpu.HBM),
        out_shape=jax.ShapeDtypeStruct((_VOCAB, _ROW_PITCH), jnp.float32),
        scratch_shapes=[
            pltpu.VMEM((_T_SLOTS, _T_TILE, _ROW_PITCH), jnp.float32),
            pltpu.SemaphoreType.DMA((_T_SLOTS,)),
        ],
    )(emb_t)


def _sc_gather_mean(idx_grouped, table_padded):
    mesh = plsc.VectorSubcoreMesh(core_axis_name="c", subcore_axis_name="s")

    @functools.partial(
        pl.kernel,
        mesh=mesh,
        out_type=jax.ShapeDtypeStruct((_BATCH, _EMBED), jnp.float32),
        compiler_params=pltpu.CompilerParams(use_tc_tiling_on_sc=False),
        scratch_types=[
            pltpu.VMEM((_N_CHUNKS, _IDX_CHUNK), jnp.int32),
            pltpu.VMEM((_IDX_PER_W, _ROW_PITCH), jnp.float32),
            pltpu.VMEM((_B_PER_W, _EMBED), jnp.float32),
            pltpu.SemaphoreType.DMA,
        ],
    )
    def body(idx_hbm, table_hbm, x_hbm, idx_v, rows_v, out_v, sem):
        wid = lax.axis_index("s") * _NC + lax.axis_index("c")
        pltpu.sync_copy(idx_hbm.at[wid], idx_v)
        copies = [
            pltpu.async_copy(
                table_hbm.at[idx_v.at[j]],
                rows_v.at[pl.ds(j * _IDX_CHUNK, _IDX_CHUNK)],
                sem,
            )
            for j in range(_N_CHUNKS)
        ]
        for c in copies:
            c.wait()

        scale = jnp.float32(1.0 / _CTX)

        def accum(i, carry):
            base = i * _CTX
            acc0 = rows_v[base, pl.ds(0, _LANES)]
            acc1 = rows_v[base, pl.ds(_LANES, _LANES)]
            for j in range(1, _CTX):
                acc0 = acc0 + rows_v[base + j, pl.ds(0, _LANES)]
                acc1 = acc1 + rows_v[base + j, pl.ds(_LANES, _LANES)]
            out_v[i, pl.ds(0, _LANES)] = acc0 * scale
            out_v[i, pl.ds(_LANES, _LANES)] = acc1 * scale
            return carry

        lax.fori_loop(0, _B_PER_W, accum, 0)
        pltpu.sync_copy(out_v, x_hbm.at[pl.ds(wid * _B_PER_W, _B_PER_W)])

    return body(idx_grouped, table_padded)


_TILE_V = 2048


def _mm_body(x_ref, wt_ref, o_ref):
    o_ref[...] = lax.dot_general(
        wt_ref[...], x_ref[...],
        (((0,), (1,)), ((), ())),
        preferred_element_type=jnp.float32,
    )


def _tc_project(x1, Wb):
    k = x1.shape[1]
    out_t = pl.pallas_call(
        _mm_body,
        grid=(pl.cdiv(_VOCAB, _TILE_V),),
        in_specs=[
            pl.BlockSpec((_BATCH, k), lambda i: (0, 0)),
            pl.BlockSpec((k, _TILE_V), lambda i: (0, i)),
        ],
        out_specs=pl.BlockSpec((_TILE_V, _BATCH), lambda i: (i, 0)),
        out_shape=jax.ShapeDtypeStruct((_VOCAB, _BATCH), jnp.float32),
    )(x1, Wb)
    return out_t.T


def kernel(inputs, emb_table, W, b):
    idx_grouped = inputs.astype(jnp.int32).reshape(_NW, _N_CHUNKS, _IDX_CHUNK)
    Wb = jnp.concatenate([W.T, b[None, :]], axis=0)
    x = _sc_gather_mean(idx_grouped, _stage_table(emb_table.T))
    x1 = jnp.concatenate([x, jnp.ones((_BATCH, 1), jnp.float32)], axis=1)
    return _tc_project(x1, Wb)


# speedup vs baseline: 1.1247x; 1.0888x over previous
import functools

import jax
import jax.numpy as jnp
from jax import lax
from jax.experimental import pallas as pl
from jax.experimental.pallas import tpu as pltpu
from jax.experimental.pallas import tpu_sc as plsc

_VOCAB = 100000
_EMBED = 32
_BATCH = 1024
_CTX = 20

_NC = 2
_NS = 16
_NW = _NC * _NS
_B_PER_W = _BATCH // _NW
_IDX_PER_W = _B_PER_W * _CTX
_IDX_CHUNK = 128
_N_CHUNKS = _IDX_PER_W // _IDX_CHUNK
_LANES = 16


_ROW_PITCH = 128
_T_TILE = 2048


_T_STEPS = pl.cdiv(_VOCAB, _T_TILE)
_T_TAIL = _VOCAB - (_T_STEPS - 1) * _T_TILE
_T_SLOTS = 4


def _pad_body(in_ref, o_hbm, buf, sem):
    i = pl.program_id(0)
    slot = lax.rem(i, _T_SLOTS)

    def _cp(step, slot_, rows):
        return pltpu.make_async_copy(
            buf.at[slot_, pl.ds(0, rows)],
            o_hbm.at[pl.ds(step * _T_TILE, rows)],
            sem.at[slot_],
        )

    @pl.when(i >= _T_SLOTS)
    def _():
        _cp(i - _T_SLOTS, slot, _T_TILE).wait()

    buf[slot, :, pl.ds(0, _EMBED)] = lax.transpose(
        in_ref[:, pl.ds(i * _T_TILE, _T_TILE)], (1, 0))

    @pl.when(i < _T_STEPS - 1)
    def _():
        _cp(i, slot, _T_TILE).start()

    @pl.when(i == _T_STEPS - 1)
    def _():
        _cp(i, slot, _T_TAIL).start()
        for back in range(_T_SLOTS - 1, 0, -1):
            _cp(i - back, lax.rem(i - back, _T_SLOTS), _T_TILE).wait()
        _cp(i, slot, _T_TAIL).wait()


def _stage_table(emb_t):
    return pl.pallas_call(
        _pad_body,
        grid=(_T_STEPS,),
        in_specs=[pl.BlockSpec((_EMBED, _VOCAB), lambda i: (0, 0))],
        out_specs=pl.BlockSpec(memory_space=pltpu.HBM),
        out_shape=jax.ShapeDtypeStruct((_VOCAB, _ROW_PITCH), jnp.float32),
        scratch_shapes=[
            pltpu.VMEM((_T_SLOTS, _T_TILE, _ROW_PITCH), jnp.float32),
            pltpu.SemaphoreType.DMA((_T_SLOTS,)),
        ],
    )(emb_t)


def _sc_gather_mean(idx_grouped, table_padded):
    mesh = plsc.VectorSubcoreMesh(core_axis_name="c", subcore_axis_name="s")

    @functools.partial(
        pl.kernel,
        mesh=mesh,
        out_type=jax.ShapeDtypeStruct((_BATCH, _EMBED), jnp.float32),
        compiler_params=pltpu.CompilerParams(use_tc_tiling_on_sc=False),
        scratch_types=[
            pltpu.VMEM((_N_CHUNKS, _IDX_CHUNK), jnp.int32),
            pltpu.VMEM((_IDX_PER_W, _ROW_PITCH), jnp.float32),
            pltpu.VMEM((_B_PER_W, _EMBED), jnp.float32),
            pltpu.SemaphoreType.DMA,
        ],
    )
    def body(idx_hbm, table_hbm, x_hbm, idx_v, rows_v, out_v, sem):
        wid = lax.axis_index("s") * _NC + lax.axis_index("c")
        pltpu.sync_copy(idx_hbm.at[wid], idx_v)
        copies = [
            pltpu.async_copy(
                table_hbm.at[idx_v.at[j]],
                rows_v.at[pl.ds(j * _IDX_CHUNK, _IDX_CHUNK)],
                sem,
            )
            for j in range(_N_CHUNKS)
        ]
        for c in copies:
            c.wait()

        scale = jnp.float32(1.0 / _CTX)

        def accum(i, carry):
            base = i * _CTX
            acc0 = rows_v[base, pl.ds(0, _LANES)]
            acc1 = rows_v[base, pl.ds(_LANES, _LANES)]
            for j in range(1, _CTX):
                acc0 = acc0 + rows_v[base + j, pl.ds(0, _LANES)]
                acc1 = acc1 + rows_v[base + j, pl.ds(_LANES, _LANES)]
            out_v[i, pl.ds(0, _LANES)] = acc0 * scale
            out_v[i, pl.ds(_LANES, _LANES)] = acc1 * scale
            return carry

        lax.fori_loop(0, _B_PER_W, accum, 0)
        pltpu.sync_copy(out_v, x_hbm.at[pl.ds(wid * _B_PER_W, _B_PER_W)])

    return body(idx_grouped, table_padded)


_TILE_V = 2048


def _mm_body(x_ref, wt_ref, o_ref):
    o_ref[...] = lax.dot_general(
        wt_ref[...], x_ref[...],
        (((0,), (1,)), ((), ())),
        preferred_element_type=jnp.float32,
    )


def _tc_project(x1, Wb):
    k = x1.shape[1]
    out_t = pl.pallas_call(
        _mm_body,
        grid=(pl.cdiv(_VOCAB, _TILE_V),),
        in_specs=[
            pl.BlockSpec((_BATCH, k), lambda i: (0, 0)),
            pl.BlockSpec((k, _TILE_V), lambda i: (0, i)),
        ],
        out_specs=pl.BlockSpec((_TILE_V, _BATCH), lambda i: (i, 0)),
        out_shape=jax.ShapeDtypeStruct((_VOCAB, _BATCH), jnp.float32),
    )(x1, Wb)
    return out_t.T


def kernel(inputs, emb_table, W, b):
    idx_grouped = inputs.astype(jnp.int32).reshape(_NW, _N_CHUNKS, _IDX_CHUNK)
    Wb = jnp.concatenate([W.T, b[None, :]], axis=0)
    x = _sc_gather_mean(idx_grouped, _stage_table(emb_table.T))
    x1 = jnp.concatenate([x, jnp.ones((_BATCH, 1), jnp.float32)], axis=1)
    return _tc_project(x1, Wb)
